# Initial kernel scaffold; baseline (speedup 1.0000x reference)
import jax
import jax.numpy as jnp
from jax.experimental import pallas as pl


def kernel(X, pool_idx):
    raise NotImplementedError("write your pallas kernel here")


# speedup vs baseline: 6.2085x; 6.2085x over previous
import functools
import jax
import jax.numpy as jnp
from jax import lax
from jax.experimental import pallas as pl
from jax.experimental.pallas import tpu as pltpu
from jax.experimental.pallas import tpu_sc as plsc

_K = 80
_NBUF = 3


@functools.partial(jax.jit, static_argnames=("n", "m", "d"))
def _pool(x, idx0, idx1, n, m, d):
    info = plsc.get_sparse_core_info()
    nc, ns, lanes = info.num_cores, info.num_subcores, info.num_lanes
    nw = nc * ns
    k = _K
    nbuf = _NBUF
    t_total = m // k
    n_max = -(-t_total // nw)
    n_grp = -(-n_max // nbuf)
    vecs = d // lanes

    mesh = plsc.VectorSubcoreMesh(core_axis_name="c", subcore_axis_name="s")

    @functools.partial(
        pl.kernel,
        out_type=jax.ShapeDtypeStruct((n + m, d), jnp.float32),
        mesh=mesh,
        scratch_types=(
            [pltpu.VMEM((k, d), jnp.float32) for _ in range(2 * nbuf)]
            + [pltpu.VMEM((n_max * k,), jnp.int32) for _ in range(2)]
            + [pltpu.SemaphoreType.DMA for _ in range(3 * nbuf)]
        ),
    )
    def sc_kernel(x_hbm, i0_hbm, i1_hbm, out_hbm, *scr):
        buf_a = scr[:nbuf]
        buf_b = scr[nbuf:2 * nbuf]
        i0_v, i1_v = scr[2 * nbuf], scr[2 * nbuf + 1]
        sem_a = scr[2 * nbuf + 2:2 * nbuf + 2 + nbuf]
        sem_b = scr[2 * nbuf + 2 + nbuf:2 * nbuf + 2 + 2 * nbuf]
        sem_s = scr[2 * nbuf + 2 + 2 * nbuf:]

        wid = lax.axis_index("s") * nc + lax.axis_index("c")
        t0 = wid * t_total // nw
        t1 = (wid + 1) * t_total // nw
        n_loc = t1 - t0

        def avg_inplace(ba, bb):
            def row_body(r, c):
                for j in range(vecs):
                    sl = pl.ds(j * lanes, lanes)
                    ba[r, sl] = (ba[r, sl] + bb[r, sl]) * 0.5
                return c
            lax.fori_loop(0, k, row_body, 0)

        def wait_store(b):
            pltpu.make_async_copy(buf_a[b], out_hbm.at[pl.ds(0, k)],
                                  sem_s[b]).wait()

        def copy_grp(g, carry):
            for b in range(nbuf):
                j = g * nbuf + b
                t = t0 + j

                @pl.when(jnp.logical_and(g > 0, (g - 1) * nbuf + b < n_loc))
                def _():
                    wait_store(b)

                @pl.when(j < n_loc)
                def _():
                    pltpu.async_copy(x_hbm.at[pl.ds(t * k, k)], buf_a[b],
                                     sem_a[b])
            for b in range(nbuf):
                j = g * nbuf + b
                t = t0 + j

                @pl.when(j < n_loc)
                def _():
                    pltpu.make_async_copy(x_hbm.at[pl.ds(0, k)], buf_a[b],
                                          sem_a[b]).wait()
                    pltpu.async_copy(buf_a[b], out_hbm.at[pl.ds(t * k, k)],
                                     sem_s[b])
            return carry

        lax.fori_loop(0, n_grp, copy_grp, 0)
        for b in range(nbuf):
            @pl.when((n_grp - 1) * nbuf + b < n_loc)
            def _():
                wait_store(b)

        pltpu.sync_copy(i0_hbm.at[pl.ds(t0 * k, n_max * k)], i0_v)
        pltpu.sync_copy(i1_hbm.at[pl.ds(t0 * k, n_max * k)], i1_v)

        def pool_grp(g, carry):
            for b in range(nbuf):
                j = g * nbuf + b

                @pl.when(jnp.logical_and(g > 0, (g - 1) * nbuf + b < n_loc))
                def _():
                    wait_store(b)

                @pl.when(j < n_loc)
                def _():
                    pltpu.async_copy(x_hbm.at[i0_v.at[pl.ds(j * k, k)]],
                                     buf_a[b], sem_a[b])
                    pltpu.async_copy(x_hbm.at[i1_v.at[pl.ds(j * k, k)]],
                                     buf_b[b], sem_b[b])
            for b in range(nbuf):
                j = g * nbuf + b
                t = t0 + j

                @pl.when(j < n_loc)
                def _():
                    pltpu.make_async_copy(x_hbm.at[i0_v.at[pl.ds(0, k)]],
                                          buf_a[b], sem_a[b]).wait()
                    pltpu.make_async_copy(x_hbm.at[i1_v.at[pl.ds(0, k)]],
                                          buf_b[b], sem_b[b]).wait()
                    avg_inplace(buf_a[b], buf_b[b])
                    pltpu.async_copy(buf_a[b],
                                     out_hbm.at[pl.ds(n + t * k, k)],
                                     sem_s[b])
            return carry

        lax.fori_loop(0, n_grp, pool_grp, 0)
        for b in range(nbuf):
            @pl.when((n_grp - 1) * nbuf + b < n_loc)
            def _():
                wait_store(b)

    return sc_kernel(x, idx0, idx1)


def kernel(X, pool_idx):
    n, d = X.shape
    m = pool_idx.shape[1]
    idx = pool_idx[0].astype(jnp.int32)
    return _pool(X, idx[:, 0], idx[:, 1], n, m, d)
